# baseline (speedup 1.0000x reference)
import jax
import jax.numpy as jnp
from jax import lax
from jax.experimental import pallas as pl
from jax.experimental.pallas import tpu as pltpu
from jax.experimental.pallas import tpu_sc as plsc

N, E, B = 10000, 160000, 16
DV, DE, DG = 256, 16, 64
HE, OE = 512, 256
HV, OV = 512, 256
HG, OG = 512, 128

VBLK = 400
EBLK = 640

_f32 = jnp.float32
_i32 = jnp.int32

_NC, _NS, _LL = 2, 16, 16
_NW = _NC * _NS


def _full(shape):
    return pl.BlockSpec(shape, lambda i: tuple(0 for _ in shape))


def _k2_body(vb, bb, g, A, Bm, Av, Cg, Cv, be1, bv1, P_o, Q_o, V1_o):
    G1 = jnp.dot(g[:, :], Cg[:, :], preferred_element_type=_f32) + be1[:, :]
    G2 = jnp.dot(g[:, :], Cv[:, :], preferred_element_type=_f32) + bv1[:, :]
    bcol = bb[:, :]
    oh = (bcol == lax.broadcasted_iota(_i32, (VBLK, B), 1).astype(_f32)).astype(_f32)
    P_o[:, :] = (jnp.dot(vb[:, :], A[:, :], preferred_element_type=_f32)
                 + jnp.dot(oh, G1, preferred_element_type=_f32))
    Q_o[:, :] = jnp.dot(vb[:, :], Bm[:, :], preferred_element_type=_f32)
    V1_o[:, :] = (jnp.dot(vb[:, :], Av[:, :], preferred_element_type=_f32)
                  + jnp.dot(oh, G2, preferred_element_type=_f32))


def _k2(v_attr, batchcol, g, A, Bm, Av, Cg, Cv, be1, bv1):
    nb = N // VBLK
    return pl.pallas_call(
        _k2_body,
        grid=(nb,),
        in_specs=[
            pl.BlockSpec((VBLK, DV), lambda i: (i, 0)),
            pl.BlockSpec((VBLK, 1), lambda i: (i, 0)),
            _full((B, DG)), _full((DV, HE)), _full((DV, HE)), _full((DV, HV)),
            _full((DG, HE)), _full((DG, HV)), _full((1, HE)), _full((1, HV)),
        ],
        out_specs=[
            pl.BlockSpec((VBLK, HE), lambda i: (i, 0)),
            pl.BlockSpec((VBLK, HE), lambda i: (i, 0)),
            pl.BlockSpec((VBLK, HV), lambda i: (i, 0)),
        ],
        out_shape=[
            jax.ShapeDtypeStruct((N, HE), _f32),
            jax.ShapeDtypeStruct((N, HE), _f32),
            jax.ShapeDtypeStruct((N, HV), _f32),
        ],
    )(v_attr, batchcol, g, A, Bm, Av, Cg, Cv, be1, bv1)


_EW = E // _NW
_ACH = 40
_ANCH = _EW // _ACH


def _sc_gather_add(P2, Q, src, dst):
    def body(p_hbm, q_hbm, src_hbm, dst_hbm, zg_hbm,
             sidx, didx, bufp, bufq, sem1, sem2):
        wid = lax.axis_index("s") * _NC + lax.axis_index("c")
        base = wid * _EW

        def chunk(ci, carry):
            off = base + ci * _ACH
            pltpu.sync_copy(src_hbm.at[pl.ds(off, _ACH)], sidx)
            pltpu.sync_copy(dst_hbm.at[pl.ds(off, _ACH)], didx)
            cp = pltpu.async_copy(p_hbm.at[sidx], bufp, sem1)
            cq = pltpu.async_copy(q_hbm.at[didx], bufq, sem2)
            cp.wait()
            cq.wait()

            def row(r, c2):
                for k in range(HE // _LL):
                    sl = pl.ds(k * _LL, _LL)
                    bufp[r, sl] = bufp[r, sl] + bufq[r, sl]
                return c2
            lax.fori_loop(0, _ACH, row, 0)
            pltpu.sync_copy(bufp, zg_hbm.at[pl.ds(off, _ACH)])
            return carry
        lax.fori_loop(0, _ANCH, chunk, 0)

    kern = pl.kernel(
        body,
        out_type=jax.ShapeDtypeStruct((E, HE), _f32),
        mesh=plsc.VectorSubcoreMesh(core_axis_name="c", subcore_axis_name="s"),
        scratch_types=[
            pltpu.VMEM((_ACH,), _i32),
            pltpu.VMEM((_ACH,), _i32),
            pltpu.VMEM((_ACH, HE), _f32),
            pltpu.VMEM((_ACH, HE), _f32),
            pltpu.SemaphoreType.DMA,
            pltpu.SemaphoreType.DMA,
        ],
    )
    return kern(P2, Q, src, dst)


def _k3_body(zb, eb, Ce, We2, be2, out_o):
    h = jnp.maximum(zb[:, :] + jnp.dot(eb[:, :], Ce[:, :], preferred_element_type=_f32), 0.0)
    out_o[:, :] = jnp.dot(h, We2[:, :], preferred_element_type=_f32) + be2[:, :]


def _k3(Zg, e_attr, Ce, We2, be2):
    nb = E // EBLK
    return pl.pallas_call(
        _k3_body,
        grid=(nb,),
        in_specs=[
            pl.BlockSpec((EBLK, HE), lambda i: (i, 0)),
            pl.BlockSpec((EBLK, DE), lambda i: (i, 0)),
            _full((DE, HE)), _full((HE, OE)), _full((1, OE)),
        ],
        out_specs=pl.BlockSpec((EBLK, OE), lambda i: (i, 0)),
        out_shape=jax.ShapeDtypeStruct((E, OE), _f32),
    )(Zg, e_attr, Ce, We2, be2)


EBLK6 = 2000
_NB6 = E // EBLK6
_NCP = N // VBLK
_NCH_S = 4
_NCH_M = 2


def _k6a_body(srcb, eb, s1o, s2o, s3o, s4o, c1o, c2o,
              s1, s2, s3, s4, cc1, cc2):
    pid = pl.program_id(0)

    @pl.when(pid == 0)
    def _init():
        for o in (s1, s2, s3, s4):
            o[:, :] = jnp.zeros((N, OE), _f32)
        for o in (cc1, cc2):
            o[:, :] = jnp.zeros((N, 16), _f32)

    ss = (s1, s2, s3, s4)
    cs = (cc1, cc2)
    step = EBLK6 // _NCH_S

    @pl.when(pid < _NB6)
    def _acc():
        def edge(i, carry):
            for c in range(_NCH_S):
                v = srcb[0, 0, c * step + i]
                row = eb[pl.ds(c * step + i, 1), :]
                ss[c][pl.ds(v, 1), :] += row
                cs[c % 2][pl.ds(v, 1), :] += 1.0
            return carry
        lax.fori_loop(0, step, edge, 0)

    @pl.when(pid >= _NB6)
    def _copy():
        r0 = (pid - _NB6) * VBLK
        for o, a in zip((s1o, s2o, s3o, s4o), ss):
            o[:, :] = a[pl.ds(r0, VBLK), :]
        for o, a in zip((c1o, c2o), cs):
            o[:, :] = a[pl.ds(r0, VBLK), :]


def _k6a(e_out, src3):
    return pl.pallas_call(
        _k6a_body,
        grid=(_NB6 + _NCP,),
        in_specs=[
            pl.BlockSpec((1, 1, EBLK6),
                         lambda i: (jnp.minimum(i, _NB6 - 1), 0, 0),
                         memory_space=pltpu.SMEM),
            pl.BlockSpec((EBLK6, OE), lambda i: (jnp.minimum(i, _NB6 - 1), 0)),
        ],
        out_specs=[pl.BlockSpec((VBLK, OE), lambda i: (jnp.maximum(i - _NB6, 0), 0))] * _NCH_S
        + [pl.BlockSpec((VBLK, 16), lambda i: (jnp.maximum(i - _NB6, 0), 0))] * 2,
        out_shape=[jax.ShapeDtypeStruct((N, OE), _f32)] * _NCH_S
        + [jax.ShapeDtypeStruct((N, 16), _f32)] * 2,
        scratch_shapes=[pltpu.VMEM((N, OE), _f32)] * _NCH_S
        + [pltpu.VMEM((N, 16), _f32)] * 2,
    )(src3, e_out)


def _k6b_body(srcb, eb, m1o, m2o, x1o, x2o, m1, m2, x1, x2):
    pid = pl.program_id(0)

    @pl.when(pid == 0)
    def _init():
        for o in (m1, m2):
            o[:, :] = jnp.full((N, OE), jnp.inf, _f32)
        for o in (x1, x2):
            o[:, :] = jnp.full((N, OE), -jnp.inf, _f32)

    ms = (m1, m2)
    xs = (x1, x2)
    step = EBLK6 // _NCH_M

    @pl.when(pid < _NB6)
    def _acc():
        def edge(i, carry):
            for c in range(_NCH_M):
                v = srcb[0, 0, c * step + i]
                row = eb[pl.ds(c * step + i, 1), :]
                ms[c][pl.ds(v, 1), :] = jnp.minimum(ms[c][pl.ds(v, 1), :], row)
                xs[c][pl.ds(v, 1), :] = jnp.maximum(xs[c][pl.ds(v, 1), :], row)
            return carry
        lax.fori_loop(0, step, edge, 0)

    @pl.when(pid >= _NB6)
    def _copy():
        r0 = (pid - _NB6) * VBLK
        for o, a in zip((m1o, m2o, x1o, x2o), (m1, m2, x1, x2)):
            o[:, :] = a[pl.ds(r0, VBLK), :]


def _k6b(e_out, src3):
    return pl.pallas_call(
        _k6b_body,
        grid=(_NB6 + _NCP,),
        in_specs=[
            pl.BlockSpec((1, 1, EBLK6),
                         lambda i: (jnp.minimum(i, _NB6 - 1), 0, 0),
                         memory_space=pltpu.SMEM),
            pl.BlockSpec((EBLK6, OE), lambda i: (jnp.minimum(i, _NB6 - 1), 0)),
        ],
        out_specs=[pl.BlockSpec((VBLK, OE), lambda i: (jnp.maximum(i - _NB6, 0), 0))] * (2 * _NCH_M),
        out_shape=[jax.ShapeDtypeStruct((N, OE), _f32)] * (2 * _NCH_M),
        scratch_shapes=[pltpu.VMEM((N, OE), _f32)] * (2 * _NCH_M),
    )(src3, e_out)


def _k4_body(v1b, m1b, m2b, s1b, s2b, s3b, s4b, x1b, x2b,
             c1b, c2b, Wmn, Wme, Wsm, Wmx, Wv2, bv2, out_o):
    c = c1b[:, 0:1] + c2b[:, 0:1]
    sb = s1b[:, :] + s2b[:, :] + s3b[:, :] + s4b[:, :]
    mnb = jnp.minimum(m1b[:, :], m2b[:, :])
    mxb = jnp.maximum(x1b[:, :], x2b[:, :])
    has = c > 0.0
    mn = jnp.where(has, mnb, 0.0)
    mx = jnp.where(has, mxb, 0.0)
    mean = sb * (1.0 / jnp.maximum(c, 1.0))
    z = (v1b[:, :]
         + jnp.dot(mn, Wmn[:, :], preferred_element_type=_f32)
         + jnp.dot(mean, Wme[:, :], preferred_element_type=_f32)
         + jnp.dot(sb, Wsm[:, :], preferred_element_type=_f32)
         + jnp.dot(mx, Wmx[:, :], preferred_element_type=_f32))
    out_o[:, :] = jnp.dot(jnp.maximum(z, 0.0), Wv2[:, :], preferred_element_type=_f32) + bv2[:, :]


def _k4(V1, MNs, Ss, MXs, CNTs, Wmn, Wme, Wsm, Wmx, Wv2, bv2):
    nb = N // VBLK
    return pl.pallas_call(
        _k4_body,
        grid=(nb,),
        in_specs=[
            pl.BlockSpec((VBLK, HV), lambda i: (i, 0)),
            pl.BlockSpec((VBLK, OE), lambda i: (i, 0)),
            pl.BlockSpec((VBLK, OE), lambda i: (i, 0)),
            pl.BlockSpec((VBLK, OE), lambda i: (i, 0)),
            pl.BlockSpec((VBLK, OE), lambda i: (i, 0)),
            pl.BlockSpec((VBLK, OE), lambda i: (i, 0)),
            pl.BlockSpec((VBLK, OE), lambda i: (i, 0)),
            pl.BlockSpec((VBLK, OE), lambda i: (i, 0)),
            pl.BlockSpec((VBLK, OE), lambda i: (i, 0)),
            pl.BlockSpec((VBLK, 16), lambda i: (i, 0)),
            pl.BlockSpec((VBLK, 16), lambda i: (i, 0)),
            _full((OE, HV)), _full((OE, HV)), _full((OE, HV)), _full((OE, HV)),
            _full((HV, OV)), _full((1, OV)),
        ],
        out_specs=pl.BlockSpec((VBLK, OV), lambda i: (i, 0)),
        out_shape=jax.ShapeDtypeStruct((N, OV), _f32),
    )(V1, MNs[0], MNs[1], Ss[0], Ss[1], Ss[2], Ss[3], MXs[0], MXs[1],
      CNTs[0], CNTs[1], Wmn, Wme, Wsm, Wmx, Wv2, bv2)


def _k5_body(m1b, m2b, s1b, s2b, s3b, s4b, x1b, x2b, c1b, c2b,
             vob, bcolb, b3b, gg, wgg, wem, wee, wes, wex,
             wvm, wve, wvs, wvx, wg2, bg1r, bg2r, out_o,
             emin_s, emax_s, esum_s, ecnt_s, vmin_s, vmax_s, vsum_s, vcnt_s):
    mnb = jnp.minimum(m1b[:, :], m2b[:, :])
    mxb = jnp.maximum(x1b[:, :], x2b[:, :])
    sb = s1b[:, :] + s2b[:, :] + s3b[:, :] + s4b[:, :]
    cb = c1b[:, :] + c2b[:, :]
    pid = pl.program_id(0)
    nb = pl.num_programs(0)

    @pl.when(pid == 0)
    def _init():
        emin_s[:, :] = jnp.full((B, OE), jnp.inf, _f32)
        emax_s[:, :] = jnp.full((B, OE), -jnp.inf, _f32)
        esum_s[:, :] = jnp.zeros((B, OE), _f32)
        ecnt_s[:, :] = jnp.zeros((B, 16), _f32)
        vmin_s[:, :] = jnp.full((B, OV), jnp.inf, _f32)
        vmax_s[:, :] = jnp.full((B, OV), -jnp.inf, _f32)
        vsum_s[:, :] = jnp.zeros((B, OV), _f32)
        vcnt_s[:, :] = jnp.zeros((B, 16), _f32)

    bcol = bcolb[:, :]
    brow = b3b[0, :, :]
    ohT = (brow == lax.broadcasted_iota(_i32, (B, VBLK), 0)).astype(_f32)
    mn = mnb
    mx = mxb
    s = sb
    vo = vob[:, :]
    ones = jnp.full((VBLK, 16), 1.0, _f32)

    esum_s[:, :] = esum_s[:, :] + jnp.dot(ohT, s, preferred_element_type=_f32)
    ecnt_s[:, :] = ecnt_s[:, :] + jnp.dot(ohT, cb, preferred_element_type=_f32)
    vsum_s[:, :] = vsum_s[:, :] + jnp.dot(ohT, vo, preferred_element_type=_f32)
    vcnt_s[:, :] = vcnt_s[:, :] + jnp.dot(ohT, ones, preferred_element_type=_f32)

    for j in range(B):
        mv = bcol == float(j)
        emin_s[j:j + 1, :] = jnp.minimum(
            emin_s[j:j + 1, :], jnp.min(jnp.where(mv, mn, jnp.inf), axis=0, keepdims=True))
        emax_s[j:j + 1, :] = jnp.maximum(
            emax_s[j:j + 1, :], jnp.max(jnp.where(mv, mx, -jnp.inf), axis=0, keepdims=True))
        vmin_s[j:j + 1, :] = jnp.minimum(
            vmin_s[j:j + 1, :], jnp.min(jnp.where(mv, vo, jnp.inf), axis=0, keepdims=True))
        vmax_s[j:j + 1, :] = jnp.maximum(
            vmax_s[j:j + 1, :], jnp.max(jnp.where(mv, vo, -jnp.inf), axis=0, keepdims=True))

    @pl.when(pid == nb - 1)
    def _final():
        ec = ecnt_s[:, 0:1]
        vc = vcnt_s[:, 0:1]
        emn = jnp.where(ec > 0.0, emin_s[:, :], 0.0)
        emx = jnp.where(ec > 0.0, emax_s[:, :], 0.0)
        eme = esum_s[:, :] * (1.0 / jnp.maximum(ec, 1.0))
        vmn = jnp.where(vc > 0.0, vmin_s[:, :], 0.0)
        vmx = jnp.where(vc > 0.0, vmax_s[:, :], 0.0)
        vme = vsum_s[:, :] * (1.0 / jnp.maximum(vc, 1.0))
        dot = lambda a, w: jnp.dot(a, w[:, :], preferred_element_type=_f32)
        z = (dot(gg[:, :], wgg) + dot(emn, wem) + dot(eme, wee)
             + dot(esum_s[:, :], wes) + dot(emx, wex) + dot(vmn, wvm)
             + dot(vme, wve) + dot(vsum_s[:, :], wvs) + dot(vmx, wvx)
             + bg1r[:, :])
        out_o[:, :] = dot(jnp.maximum(z, 0.0), wg2) + bg2r[:, :]


def _k5(MNs, Ss, MXs, CNTs, v_out, batchcol, batch3, g, Wg_slices, bg1, Wg2, bg2):
    nb = N // VBLK
    return pl.pallas_call(
        _k5_body,
        grid=(nb,),
        in_specs=[
            pl.BlockSpec((VBLK, OE), lambda i: (i, 0)),
            pl.BlockSpec((VBLK, OE), lambda i: (i, 0)),
            pl.BlockSpec((VBLK, OE), lambda i: (i, 0)),
            pl.BlockSpec((VBLK, OE), lambda i: (i, 0)),
            pl.BlockSpec((VBLK, OE), lambda i: (i, 0)),
            pl.BlockSpec((VBLK, OE), lambda i: (i, 0)),
            pl.BlockSpec((VBLK, OE), lambda i: (i, 0)),
            pl.BlockSpec((VBLK, OE), lambda i: (i, 0)),
            pl.BlockSpec((VBLK, 16), lambda i: (i, 0)),
            pl.BlockSpec((VBLK, 16), lambda i: (i, 0)),
            pl.BlockSpec((VBLK, OV), lambda i: (i, 0)),
            pl.BlockSpec((VBLK, 1), lambda i: (i, 0)),
            pl.BlockSpec((1, 1, VBLK), lambda i: (i, 0, 0)),
            _full((B, DG)),
            _full((DG, HG)), _full((OE, HG)), _full((OE, HG)), _full((OE, HG)),
            _full((OE, HG)), _full((OV, HG)), _full((OV, HG)), _full((OV, HG)),
            _full((OV, HG)), _full((HG, OG)), _full((1, HG)), _full((1, OG)),
        ],
        out_specs=pl.BlockSpec((B, OG), lambda i: (0, 0)),
        out_shape=jax.ShapeDtypeStruct((B, OG), _f32),
        scratch_shapes=[
            pltpu.VMEM((B, OE), _f32), pltpu.VMEM((B, OE), _f32),
            pltpu.VMEM((B, OE), _f32), pltpu.VMEM((B, 16), _f32),
            pltpu.VMEM((B, OV), _f32), pltpu.VMEM((B, OV), _f32),
            pltpu.VMEM((B, OV), _f32), pltpu.VMEM((B, 16), _f32),
        ],
    )(MNs[0], MNs[1], Ss[0], Ss[1], Ss[2], Ss[3], MXs[0], MXs[1],
      CNTs[0], CNTs[1], v_out, batchcol, batch3, g,
      *Wg_slices, Wg2, bg1, bg2)


def kernel(v_attr, e_attr, g, We1, be1, We2, be2, Wv1, bv1, Wv2, bv2,
           Wg1, bg1, Wg2, bg2, edgeij_pair, batch):
    src = edgeij_pair[0].astype(_i32)
    dst = edgeij_pair[1].astype(_i32)
    batch = batch.astype(_i32)
    batchcol = batch.astype(_f32).reshape(N, 1)
    batch3 = batch.reshape(N // VBLK, 1, VBLK)

    A, Bm, Ce, Cg = We1[0:256], We1[256:512], We1[512:528], We1[528:592]
    Av, Wmn, Wme, Wsm, Wmx, Cv = (Wv1[0:256], Wv1[256:512], Wv1[512:768],
                                  Wv1[768:1024], Wv1[1024:1280], Wv1[1280:1344])
    Wg_slices = (Wg1[0:64], Wg1[64:320], Wg1[320:576], Wg1[576:832],
                 Wg1[832:1088], Wg1[1088:1344], Wg1[1344:1600],
                 Wg1[1600:1856], Wg1[1856:2112])
    be1r, bv1r, be2r, bv2r = (be1.reshape(1, HE), bv1.reshape(1, HV),
                              be2.reshape(1, OE), bv2.reshape(1, OV))
    bg1r, bg2r = bg1.reshape(1, HG), bg2.reshape(1, OG)

    P2, Q, V1 = _k2(v_attr, batchcol, g, A, Bm, Av, Cg, Cv, be1r, bv1r)

    Zg = _sc_gather_add(P2, Q, src, dst)

    e_out = _k3(Zg, e_attr, Ce, We2, be2r)

    src3 = src.reshape(E // EBLK6, 1, EBLK6)
    s1, s2, s3, s4, c1, c2 = _k6a(e_out, src3)
    m1, m2, x1, x2 = _k6b(e_out, src3)
    Ss, CNTs, MNs, MXs = (s1, s2, s3, s4), (c1, c2), (m1, m2), (x1, x2)

    v_out = _k4(V1, MNs, Ss, MXs, CNTs, Wmn, Wme, Wsm, Wmx, Wv2, bv2r)

    g_out = _k5(MNs, Ss, MXs, CNTs, v_out, batchcol, batch3, g, Wg_slices,
                bg1r, Wg2, bg2r)

    return (e_out, v_out, g_out)


# speedup vs baseline: 1.1356x; 1.1356x over previous
import jax
import jax.numpy as jnp
from jax import lax
from jax.experimental import pallas as pl
from jax.experimental.pallas import tpu as pltpu
from jax.experimental.pallas import tpu_sc as plsc

N, E, B = 10000, 160000, 16
DV, DE, DG = 256, 16, 64
HE, OE = 512, 256
HV, OV = 512, 256
HG, OG = 512, 128

VBLK = 400
EBLK = 640

_f32 = jnp.float32
_i32 = jnp.int32

_NC, _NS, _LL = 2, 16, 16
_NW = _NC * _NS


def _full(shape):
    return pl.BlockSpec(shape, lambda i: tuple(0 for _ in shape))


def _k2_body(vb, bb, g, A, Bm, Av, Cg, Cv, be1, bv1, P_o, Q_o, V1_o):
    G1 = jnp.dot(g[:, :], Cg[:, :], preferred_element_type=_f32) + be1[:, :]
    G2 = jnp.dot(g[:, :], Cv[:, :], preferred_element_type=_f32) + bv1[:, :]
    bcol = bb[:, :]
    oh = (bcol == lax.broadcasted_iota(_i32, (VBLK, B), 1).astype(_f32)).astype(_f32)
    P_o[:, :] = (jnp.dot(vb[:, :], A[:, :], preferred_element_type=_f32)
                 + jnp.dot(oh, G1, preferred_element_type=_f32))
    Q_o[:, :] = jnp.dot(vb[:, :], Bm[:, :], preferred_element_type=_f32)
    V1_o[:, :] = (jnp.dot(vb[:, :], Av[:, :], preferred_element_type=_f32)
                  + jnp.dot(oh, G2, preferred_element_type=_f32))


def _k2(v_attr, batchcol, g, A, Bm, Av, Cg, Cv, be1, bv1):
    nb = N // VBLK
    return pl.pallas_call(
        _k2_body,
        grid=(nb,),
        in_specs=[
            pl.BlockSpec((VBLK, DV), lambda i: (i, 0)),
            pl.BlockSpec((VBLK, 1), lambda i: (i, 0)),
            _full((B, DG)), _full((DV, HE)), _full((DV, HE)), _full((DV, HV)),
            _full((DG, HE)), _full((DG, HV)), _full((1, HE)), _full((1, HV)),
        ],
        out_specs=[
            pl.BlockSpec((VBLK, HE), lambda i: (i, 0)),
            pl.BlockSpec((VBLK, HE), lambda i: (i, 0)),
            pl.BlockSpec((VBLK, HV), lambda i: (i, 0)),
        ],
        out_shape=[
            jax.ShapeDtypeStruct((N, HE), _f32),
            jax.ShapeDtypeStruct((N, HE), _f32),
            jax.ShapeDtypeStruct((N, HV), _f32),
        ],
    )(v_attr, batchcol, g, A, Bm, Av, Cg, Cv, be1, bv1)


_EW = E // _NW
_ACH = 40
_ANCH = _EW // _ACH


def _sc_gather_add(P2, Q, src, dst):
    def body(p_hbm, q_hbm, src_hbm, dst_hbm, zg_hbm,
             sidx, didx, bufp, bufq, sem1, sem2):
        wid = lax.axis_index("s") * _NC + lax.axis_index("c")
        base = wid * _EW

        def chunk(ci, carry):
            off = base + ci * _ACH
            pltpu.sync_copy(src_hbm.at[pl.ds(off, _ACH)], sidx)
            pltpu.sync_copy(dst_hbm.at[pl.ds(off, _ACH)], didx)
            cp = pltpu.async_copy(p_hbm.at[sidx], bufp, sem1)
            cq = pltpu.async_copy(q_hbm.at[didx], bufq, sem2)
            cp.wait()
            cq.wait()

            def row(r, c2):
                for k in range(HE // _LL):
                    sl = pl.ds(k * _LL, _LL)
                    bufp[r, sl] = bufp[r, sl] + bufq[r, sl]
                return c2
            lax.fori_loop(0, _ACH, row, 0)
            pltpu.sync_copy(bufp, zg_hbm.at[pl.ds(off, _ACH)])
            return carry
        lax.fori_loop(0, _ANCH, chunk, 0)

    kern = pl.kernel(
        body,
        out_type=jax.ShapeDtypeStruct((E, HE), _f32),
        mesh=plsc.VectorSubcoreMesh(core_axis_name="c", subcore_axis_name="s"),
        scratch_types=[
            pltpu.VMEM((_ACH,), _i32),
            pltpu.VMEM((_ACH,), _i32),
            pltpu.VMEM((_ACH, HE), _f32),
            pltpu.VMEM((_ACH, HE), _f32),
            pltpu.SemaphoreType.DMA,
            pltpu.SemaphoreType.DMA,
        ],
    )
    return kern(P2, Q, src, dst)


def _k3_body(zb, eb, Ce, We2, be2, out_o):
    h = jnp.maximum(zb[:, :] + jnp.dot(eb[:, :], Ce[:, :], preferred_element_type=_f32), 0.0)
    out_o[:, :] = jnp.dot(h, We2[:, :], preferred_element_type=_f32) + be2[:, :]


def _k3(Zg, e_attr, Ce, We2, be2):
    nb = E // EBLK
    return pl.pallas_call(
        _k3_body,
        grid=(nb,),
        in_specs=[
            pl.BlockSpec((EBLK, HE), lambda i: (i, 0)),
            pl.BlockSpec((EBLK, DE), lambda i: (i, 0)),
            _full((DE, HE)), _full((HE, OE)), _full((1, OE)),
        ],
        out_specs=pl.BlockSpec((EBLK, OE), lambda i: (i, 0)),
        out_shape=jax.ShapeDtypeStruct((E, OE), _f32),
    )(Zg, e_attr, Ce, We2, be2)


EBLK6 = 2000


def _k6_body(srcb, eb, s_o, mn_o, mx_o, c_o):
    pid = pl.program_id(0)

    @pl.when(pid == 0)
    def _init():
        s_o[:, :] = jnp.zeros((N, OE), _f32)
        mn_o[:, :] = jnp.full((N, OE), jnp.inf, _f32)
        mx_o[:, :] = jnp.full((N, OE), -jnp.inf, _f32)
        c_o[:, :] = jnp.zeros((N, 16), _f32)

    def edge(i, carry):
        v = srcb[0, 0, i]
        row = eb[pl.ds(i, 1), :]
        s_o[pl.ds(v, 1), :] += row
        mn_o[pl.ds(v, 1), :] = jnp.minimum(mn_o[pl.ds(v, 1), :], row)
        mx_o[pl.ds(v, 1), :] = jnp.maximum(mx_o[pl.ds(v, 1), :], row)
        c_o[pl.ds(v, 1), :] += 1.0
        return carry
    lax.fori_loop(0, EBLK6, edge, 0)


def _k6(e_out, src2):
    nb = E // EBLK6
    return pl.pallas_call(
        _k6_body,
        grid=(nb,),
        in_specs=[
            pl.BlockSpec((1, 1, EBLK6), lambda i: (i, 0, 0), memory_space=pltpu.SMEM),
            pl.BlockSpec((EBLK6, OE), lambda i: (i, 0)),
        ],
        out_specs=[
            pl.BlockSpec((N, OE), lambda i: (0, 0)),
            pl.BlockSpec((N, OE), lambda i: (0, 0)),
            pl.BlockSpec((N, OE), lambda i: (0, 0)),
            pl.BlockSpec((N, 16), lambda i: (0, 0)),
        ],
        out_shape=[
            jax.ShapeDtypeStruct((N, OE), _f32),
            jax.ShapeDtypeStruct((N, OE), _f32),
            jax.ShapeDtypeStruct((N, OE), _f32),
            jax.ShapeDtypeStruct((N, 16), _f32),
        ],
    )(src2, e_out)


def _k4_body(v1b, mnb, sb, mxb, cb, Wmn, Wme, Wsm, Wmx, Wv2, bv2, out_o):
    c = cb[:, 0:1]
    has = c > 0.0
    mn = jnp.where(has, mnb[:, :], 0.0)
    mx = jnp.where(has, mxb[:, :], 0.0)
    mean = sb[:, :] * (1.0 / jnp.maximum(c, 1.0))
    z = (v1b[:, :]
         + jnp.dot(mn, Wmn[:, :], preferred_element_type=_f32)
         + jnp.dot(mean, Wme[:, :], preferred_element_type=_f32)
         + jnp.dot(sb[:, :], Wsm[:, :], preferred_element_type=_f32)
         + jnp.dot(mx, Wmx[:, :], preferred_element_type=_f32))
    out_o[:, :] = jnp.dot(jnp.maximum(z, 0.0), Wv2[:, :], preferred_element_type=_f32) + bv2[:, :]


def _k4(V1, MN, S, MX, CNT, Wmn, Wme, Wsm, Wmx, Wv2, bv2):
    nb = N // VBLK
    return pl.pallas_call(
        _k4_body,
        grid=(nb,),
        in_specs=[
            pl.BlockSpec((VBLK, HV), lambda i: (i, 0)),
            pl.BlockSpec((VBLK, OE), lambda i: (i, 0)),
            pl.BlockSpec((VBLK, OE), lambda i: (i, 0)),
            pl.BlockSpec((VBLK, OE), lambda i: (i, 0)),
            pl.BlockSpec((VBLK, 16), lambda i: (i, 0)),
            _full((OE, HV)), _full((OE, HV)), _full((OE, HV)), _full((OE, HV)),
            _full((HV, OV)), _full((1, OV)),
        ],
        out_specs=pl.BlockSpec((VBLK, OV), lambda i: (i, 0)),
        out_shape=jax.ShapeDtypeStruct((N, OV), _f32),
    )(V1, MN, S, MX, CNT, Wmn, Wme, Wsm, Wmx, Wv2, bv2)


def _k5_body(mnb, sb, mxb, cb, vob, bcolb, b3b, bsm, gg, wgg, wem, wee, wes, wex,
             wvm, wve, wvs, wvx, wg2, bg1r, bg2r, out_o,
             emin_s, emax_s, esum_s, ecnt_s, vmin_s, vmax_s, vsum_s, vcnt_s):
    pid = pl.program_id(0)
    nb = pl.num_programs(0)

    @pl.when(pid == 0)
    def _init():
        emin_s[:, :] = jnp.full((B, OE), jnp.inf, _f32)
        emax_s[:, :] = jnp.full((B, OE), -jnp.inf, _f32)
        esum_s[:, :] = jnp.zeros((B, OE), _f32)
        ecnt_s[:, :] = jnp.zeros((B, 16), _f32)
        vmin_s[:, :] = jnp.full((B, OV), jnp.inf, _f32)
        vmax_s[:, :] = jnp.full((B, OV), -jnp.inf, _f32)
        vsum_s[:, :] = jnp.zeros((B, OV), _f32)
        vcnt_s[:, :] = jnp.zeros((B, 16), _f32)

    bcol = bcolb[:, :]
    brow = b3b[0, :, :]
    ohT = (brow == lax.broadcasted_iota(_i32, (B, VBLK), 0)).astype(_f32)
    mn = mnb[:, :]
    mx = mxb[:, :]
    s = sb[:, :]
    vo = vob[:, :]
    ones = jnp.full((VBLK, 16), 1.0, _f32)

    esum_s[:, :] = esum_s[:, :] + jnp.dot(ohT, s, preferred_element_type=_f32)
    ecnt_s[:, :] = ecnt_s[:, :] + jnp.dot(ohT, cb[:, :], preferred_element_type=_f32)
    vsum_s[:, :] = vsum_s[:, :] + jnp.dot(ohT, vo, preferred_element_type=_f32)
    vcnt_s[:, :] = vcnt_s[:, :] + jnp.dot(ohT, ones, preferred_element_type=_f32)

    blo = bsm[0, 0, 0]
    bhi = bsm[0, 0, VBLK - 1]
    for j in range(B):
        @pl.when((j >= blo) & (j <= bhi))
        def _upd(j=j):
            mv = bcol == float(j)
            emin_s[j:j + 1, :] = jnp.minimum(
                emin_s[j:j + 1, :], jnp.min(jnp.where(mv, mn, jnp.inf), axis=0, keepdims=True))
            emax_s[j:j + 1, :] = jnp.maximum(
                emax_s[j:j + 1, :], jnp.max(jnp.where(mv, mx, -jnp.inf), axis=0, keepdims=True))
            vmin_s[j:j + 1, :] = jnp.minimum(
                vmin_s[j:j + 1, :], jnp.min(jnp.where(mv, vo, jnp.inf), axis=0, keepdims=True))
            vmax_s[j:j + 1, :] = jnp.maximum(
                vmax_s[j:j + 1, :], jnp.max(jnp.where(mv, vo, -jnp.inf), axis=0, keepdims=True))

    @pl.when(pid == nb - 1)
    def _final():
        ec = ecnt_s[:, 0:1]
        vc = vcnt_s[:, 0:1]
        emn = jnp.where(ec > 0.0, emin_s[:, :], 0.0)
        emx = jnp.where(ec > 0.0, emax_s[:, :], 0.0)
        eme = esum_s[:, :] * (1.0 / jnp.maximum(ec, 1.0))
        vmn = jnp.where(vc > 0.0, vmin_s[:, :], 0.0)
        vmx = jnp.where(vc > 0.0, vmax_s[:, :], 0.0)
        vme = vsum_s[:, :] * (1.0 / jnp.maximum(vc, 1.0))
        dot = lambda a, w: jnp.dot(a, w[:, :], preferred_element_type=_f32)
        z = (dot(gg[:, :], wgg) + dot(emn, wem) + dot(eme, wee)
             + dot(esum_s[:, :], wes) + dot(emx, wex) + dot(vmn, wvm)
             + dot(vme, wve) + dot(vsum_s[:, :], wvs) + dot(vmx, wvx)
             + bg1r[:, :])
        out_o[:, :] = dot(jnp.maximum(z, 0.0), wg2) + bg2r[:, :]


def _k5(MN, S, MX, CNT, v_out, batchcol, batch3, g, Wg_slices, bg1, Wg2, bg2):
    nb = N // VBLK
    return pl.pallas_call(
        _k5_body,
        grid=(nb,),
        in_specs=[
            pl.BlockSpec((VBLK, OE), lambda i: (i, 0)),
            pl.BlockSpec((VBLK, OE), lambda i: (i, 0)),
            pl.BlockSpec((VBLK, OE), lambda i: (i, 0)),
            pl.BlockSpec((VBLK, 16), lambda i: (i, 0)),
            pl.BlockSpec((VBLK, OV), lambda i: (i, 0)),
            pl.BlockSpec((VBLK, 1), lambda i: (i, 0)),
            pl.BlockSpec((1, 1, VBLK), lambda i: (i, 0, 0)),
            pl.BlockSpec((1, 1, VBLK), lambda i: (i, 0, 0), memory_space=pltpu.SMEM),
            _full((B, DG)),
            _full((DG, HG)), _full((OE, HG)), _full((OE, HG)), _full((OE, HG)),
            _full((OE, HG)), _full((OV, HG)), _full((OV, HG)), _full((OV, HG)),
            _full((OV, HG)), _full((HG, OG)), _full((1, HG)), _full((1, OG)),
        ],
        out_specs=pl.BlockSpec((B, OG), lambda i: (0, 0)),
        out_shape=jax.ShapeDtypeStruct((B, OG), _f32),
        scratch_shapes=[
            pltpu.VMEM((B, OE), _f32), pltpu.VMEM((B, OE), _f32),
            pltpu.VMEM((B, OE), _f32), pltpu.VMEM((B, 16), _f32),
            pltpu.VMEM((B, OV), _f32), pltpu.VMEM((B, OV), _f32),
            pltpu.VMEM((B, OV), _f32), pltpu.VMEM((B, 16), _f32),
        ],
    )(MN, S, MX, CNT, v_out, batchcol, batch3, batch3, g, *Wg_slices, Wg2, bg1, bg2)


def kernel(v_attr, e_attr, g, We1, be1, We2, be2, Wv1, bv1, Wv2, bv2,
           Wg1, bg1, Wg2, bg2, edgeij_pair, batch):
    src = edgeij_pair[0].astype(_i32)
    dst = edgeij_pair[1].astype(_i32)
    batch = batch.astype(_i32)
    batchcol = batch.astype(_f32).reshape(N, 1)
    batch3 = batch.reshape(N // VBLK, 1, VBLK)

    A, Bm, Ce, Cg = We1[0:256], We1[256:512], We1[512:528], We1[528:592]
    Av, Wmn, Wme, Wsm, Wmx, Cv = (Wv1[0:256], Wv1[256:512], Wv1[512:768],
                                  Wv1[768:1024], Wv1[1024:1280], Wv1[1280:1344])
    Wg_slices = (Wg1[0:64], Wg1[64:320], Wg1[320:576], Wg1[576:832],
                 Wg1[832:1088], Wg1[1088:1344], Wg1[1344:1600],
                 Wg1[1600:1856], Wg1[1856:2112])
    be1r, bv1r, be2r, bv2r = (be1.reshape(1, HE), bv1.reshape(1, HV),
                              be2.reshape(1, OE), bv2.reshape(1, OV))
    bg1r, bg2r = bg1.reshape(1, HG), bg2.reshape(1, OG)

    P2, Q, V1 = _k2(v_attr, batchcol, g, A, Bm, Av, Cg, Cv, be1r, bv1r)

    Zg = _sc_gather_add(P2, Q, src, dst)

    e_out = _k3(Zg, e_attr, Ce, We2, be2r)

    S, MN, MX, CNT = _k6(e_out, src.reshape(E // EBLK6, 1, EBLK6))

    v_out = _k4(V1, MN, S, MX, CNT, Wmn, Wme, Wsm, Wmx, Wv2, bv2r)

    g_out = _k5(MN, S, MX, CNT, v_out, batchcol, batch3, g, Wg_slices,
                bg1r, Wg2, bg2r)

    return (e_out, v_out, g_out)


# speedup vs baseline: 1.2900x; 1.1360x over previous
import jax
import jax.numpy as jnp
from jax import lax
from jax.experimental import pallas as pl
from jax.experimental.pallas import tpu as pltpu
from jax.experimental.pallas import tpu_sc as plsc

N, E, B = 10000, 160000, 16
DV, DE, DG = 256, 16, 64
HE, OE = 512, 256
HV, OV = 512, 256
HG, OG = 512, 128

VBLK = 400
EBLK = 640

_f32 = jnp.float32
_i32 = jnp.int32

_NC, _NS, _LL = 2, 16, 16
_NW = _NC * _NS


def _full(shape):
    return pl.BlockSpec(shape, lambda i: tuple(0 for _ in shape))


def _k2_body(vb, bb, g, A, Bm, Av, Cg, Cv, be1, bv1, P_o, Q_o, V1_o):
    G1 = jnp.dot(g[:, :], Cg[:, :], preferred_element_type=_f32) + be1[:, :]
    G2 = jnp.dot(g[:, :], Cv[:, :], preferred_element_type=_f32) + bv1[:, :]
    bcol = bb[:, :]
    oh = (bcol == lax.broadcasted_iota(_i32, (VBLK, B), 1).astype(_f32)).astype(_f32)
    P_o[:, :] = (jnp.dot(vb[:, :], A[:, :], preferred_element_type=_f32)
                 + jnp.dot(oh, G1, preferred_element_type=_f32))
    Q_o[:, :] = jnp.dot(vb[:, :], Bm[:, :], preferred_element_type=_f32)
    V1_o[:, :] = (jnp.dot(vb[:, :], Av[:, :], preferred_element_type=_f32)
                  + jnp.dot(oh, G2, preferred_element_type=_f32))


def _k2(v_attr, batchcol, g, A, Bm, Av, Cg, Cv, be1, bv1):
    nb = N // VBLK
    return pl.pallas_call(
        _k2_body,
        grid=(nb,),
        in_specs=[
            pl.BlockSpec((VBLK, DV), lambda i: (i, 0)),
            pl.BlockSpec((VBLK, 1), lambda i: (i, 0)),
            _full((B, DG)), _full((DV, HE)), _full((DV, HE)), _full((DV, HV)),
            _full((DG, HE)), _full((DG, HV)), _full((1, HE)), _full((1, HV)),
        ],
        out_specs=[
            pl.BlockSpec((VBLK, HE), lambda i: (i, 0)),
            pl.BlockSpec((VBLK, HE), lambda i: (i, 0)),
            pl.BlockSpec((VBLK, HV), lambda i: (i, 0)),
        ],
        out_shape=[
            jax.ShapeDtypeStruct((N, HE), _f32),
            jax.ShapeDtypeStruct((N, HE), _f32),
            jax.ShapeDtypeStruct((N, HV), _f32),
        ],
    )(v_attr, batchcol, g, A, Bm, Av, Cg, Cv, be1, bv1)


_EW = E // _NW
_ACH = 40
_ANCH = _EW // _ACH


def _sc_gather_add(P2, Q, src, dst):
    def body(p_hbm, q_hbm, src_hbm, dst_hbm, zg_hbm,
             sidx, didx, bufp0, bufq0, bufp1, bufq1, semp0, semq0, semp1, semq1):
        wid = lax.axis_index("s") * _NC + lax.axis_index("c")
        base = wid * _EW
        pltpu.sync_copy(src_hbm.at[pl.ds(base, _EW)], sidx)
        pltpu.sync_copy(dst_hbm.at[pl.ds(base, _EW)], didx)

        bufps = (bufp0, bufp1)
        bufqs = (bufq0, bufq1)
        semps = (semp0, semp1)
        semqs = (semq0, semq1)

        def issue(ci, b):
            pltpu.async_copy(p_hbm.at[sidx.at[pl.ds(ci * _ACH, _ACH)]],
                             bufps[b], semps[b])
            pltpu.async_copy(q_hbm.at[didx.at[pl.ds(ci * _ACH, _ACH)]],
                             bufqs[b], semqs[b])

        def drain_process(ci, b):
            pltpu.make_async_copy(p_hbm.at[sidx.at[pl.ds(ci * _ACH, _ACH)]],
                                  bufps[b], semps[b]).wait()
            pltpu.make_async_copy(q_hbm.at[didx.at[pl.ds(ci * _ACH, _ACH)]],
                                  bufqs[b], semqs[b]).wait()
            bp = bufps[b]
            bq = bufqs[b]

            def row(r, c2):
                for k in range(HE // _LL):
                    sl = pl.ds(k * _LL, _LL)
                    bp[r, sl] = bp[r, sl] + bq[r, sl]
                return c2
            lax.fori_loop(0, _ACH, row, 0)
            pltpu.sync_copy(bp, zg_hbm.at[pl.ds(base + ci * _ACH, _ACH)])

        issue(0, 0)

        def chunk2(h, carry):
            ci = h * 2

            def nb0():
                issue(ci + 1, 1)
            pl.when(ci + 1 < _ANCH)(nb0)
            drain_process(ci, 0)

            def nb1():
                issue(ci + 2, 0)
            pl.when(ci + 2 < _ANCH)(nb1)

            def pr1():
                drain_process(ci + 1, 1)
            pl.when(ci + 1 < _ANCH)(pr1)
            return carry
        lax.fori_loop(0, (_ANCH + 1) // 2, chunk2, 0)

    kern = pl.kernel(
        body,
        out_type=jax.ShapeDtypeStruct((E, HE), _f32),
        mesh=plsc.VectorSubcoreMesh(core_axis_name="c", subcore_axis_name="s"),
        scratch_types=[
            pltpu.VMEM((_EW,), _i32),
            pltpu.VMEM((_EW,), _i32),
            pltpu.VMEM((_ACH, HE), _f32),
            pltpu.VMEM((_ACH, HE), _f32),
            pltpu.VMEM((_ACH, HE), _f32),
            pltpu.VMEM((_ACH, HE), _f32),
            pltpu.SemaphoreType.DMA,
            pltpu.SemaphoreType.DMA,
            pltpu.SemaphoreType.DMA,
            pltpu.SemaphoreType.DMA,
        ],
    )
    return kern(P2, Q, src, dst)


def _k3_body(zb, eb, Ce, We2, be2, out_o):
    h = jnp.maximum(zb[:, :] + jnp.dot(eb[:, :], Ce[:, :], preferred_element_type=_f32), 0.0)
    out_o[:, :] = jnp.dot(h, We2[:, :], preferred_element_type=_f32) + be2[:, :]


def _k3(Zg, e_attr, Ce, We2, be2):
    nb = E // EBLK
    return pl.pallas_call(
        _k3_body,
        grid=(nb,),
        in_specs=[
            pl.BlockSpec((EBLK, HE), lambda i: (i, 0)),
            pl.BlockSpec((EBLK, DE), lambda i: (i, 0)),
            _full((DE, HE)), _full((HE, OE)), _full((1, OE)),
        ],
        out_specs=pl.BlockSpec((EBLK, OE), lambda i: (i, 0)),
        out_shape=jax.ShapeDtypeStruct((E, OE), _f32),
    )(Zg, e_attr, Ce, We2, be2)


EBLK6 = 2000


def _k6_body(srcb, eb, s_o, mn_o, mx_o, c_o):
    pid = pl.program_id(0)

    @pl.when(pid == 0)
    def _init():
        s_o[:, :] = jnp.zeros((N, OE), _f32)
        mn_o[:, :] = jnp.full((N, OE), jnp.inf, _f32)
        mx_o[:, :] = jnp.full((N, OE), -jnp.inf, _f32)
        c_o[:, :] = jnp.zeros((N, 16), _f32)

    def edge(i, carry):
        v = srcb[0, 0, i]
        row = eb[pl.ds(i, 1), :]
        s_o[pl.ds(v, 1), :] += row
        mn_o[pl.ds(v, 1), :] = jnp.minimum(mn_o[pl.ds(v, 1), :], row)
        mx_o[pl.ds(v, 1), :] = jnp.maximum(mx_o[pl.ds(v, 1), :], row)
        c_o[pl.ds(v, 1), :] += 1.0
        return carry
    lax.fori_loop(0, EBLK6, edge, 0)


def _k6(e_out, src2):
    nb = E // EBLK6
    return pl.pallas_call(
        _k6_body,
        grid=(nb,),
        in_specs=[
            pl.BlockSpec((1, 1, EBLK6), lambda i: (i, 0, 0), memory_space=pltpu.SMEM),
            pl.BlockSpec((EBLK6, OE), lambda i: (i, 0)),
        ],
        out_specs=[
            pl.BlockSpec((N, OE), lambda i: (0, 0)),
            pl.BlockSpec((N, OE), lambda i: (0, 0)),
            pl.BlockSpec((N, OE), lambda i: (0, 0)),
            pl.BlockSpec((N, 16), lambda i: (0, 0)),
        ],
        out_shape=[
            jax.ShapeDtypeStruct((N, OE), _f32),
            jax.ShapeDtypeStruct((N, OE), _f32),
            jax.ShapeDtypeStruct((N, OE), _f32),
            jax.ShapeDtypeStruct((N, 16), _f32),
        ],
    )(src2, e_out)


def _k4_body(v1b, mnb, sb, mxb, cb, Wmn, Wme, Wsm, Wmx, Wv2, bv2, out_o):
    c = cb[:, 0:1]
    has = c > 0.0
    mn = jnp.where(has, mnb[:, :], 0.0)
    mx = jnp.where(has, mxb[:, :], 0.0)
    mean = sb[:, :] * (1.0 / jnp.maximum(c, 1.0))
    z = (v1b[:, :]
         + jnp.dot(mn, Wmn[:, :], preferred_element_type=_f32)
         + jnp.dot(mean, Wme[:, :], preferred_element_type=_f32)
         + jnp.dot(sb[:, :], Wsm[:, :], preferred_element_type=_f32)
         + jnp.dot(mx, Wmx[:, :], preferred_element_type=_f32))
    out_o[:, :] = jnp.dot(jnp.maximum(z, 0.0), Wv2[:, :], preferred_element_type=_f32) + bv2[:, :]


def _k4(V1, MN, S, MX, CNT, Wmn, Wme, Wsm, Wmx, Wv2, bv2):
    nb = N // VBLK
    return pl.pallas_call(
        _k4_body,
        grid=(nb,),
        in_specs=[
            pl.BlockSpec((VBLK, HV), lambda i: (i, 0)),
            pl.BlockSpec((VBLK, OE), lambda i: (i, 0)),
            pl.BlockSpec((VBLK, OE), lambda i: (i, 0)),
            pl.BlockSpec((VBLK, OE), lambda i: (i, 0)),
            pl.BlockSpec((VBLK, 16), lambda i: (i, 0)),
            _full((OE, HV)), _full((OE, HV)), _full((OE, HV)), _full((OE, HV)),
            _full((HV, OV)), _full((1, OV)),
        ],
        out_specs=pl.BlockSpec((VBLK, OV), lambda i: (i, 0)),
        out_shape=jax.ShapeDtypeStruct((N, OV), _f32),
    )(V1, MN, S, MX, CNT, Wmn, Wme, Wsm, Wmx, Wv2, bv2)


def _k5_body(mnb, sb, mxb, cb, vob, bcolb, b3b, bsm, gg, wgg, wem, wee, wes, wex,
             wvm, wve, wvs, wvx, wg2, bg1r, bg2r, out_o,
             emin_s, emax_s, esum_s, ecnt_s, vmin_s, vmax_s, vsum_s, vcnt_s):
    pid = pl.program_id(0)
    nb = pl.num_programs(0)

    @pl.when(pid == 0)
    def _init():
        emin_s[:, :] = jnp.full((B, OE), jnp.inf, _f32)
        emax_s[:, :] = jnp.full((B, OE), -jnp.inf, _f32)
        esum_s[:, :] = jnp.zeros((B, OE), _f32)
        ecnt_s[:, :] = jnp.zeros((B, 16), _f32)
        vmin_s[:, :] = jnp.full((B, OV), jnp.inf, _f32)
        vmax_s[:, :] = jnp.full((B, OV), -jnp.inf, _f32)
        vsum_s[:, :] = jnp.zeros((B, OV), _f32)
        vcnt_s[:, :] = jnp.zeros((B, 16), _f32)

    bcol = bcolb[:, :]
    brow = b3b[0, :, :]
    ohT = (brow == lax.broadcasted_iota(_i32, (B, VBLK), 0)).astype(_f32)
    mn = mnb[:, :]
    mx = mxb[:, :]
    s = sb[:, :]
    vo = vob[:, :]
    ones = jnp.full((VBLK, 16), 1.0, _f32)

    esum_s[:, :] = esum_s[:, :] + jnp.dot(ohT, s, preferred_element_type=_f32)
    ecnt_s[:, :] = ecnt_s[:, :] + jnp.dot(ohT, cb[:, :], preferred_element_type=_f32)
    vsum_s[:, :] = vsum_s[:, :] + jnp.dot(ohT, vo, preferred_element_type=_f32)
    vcnt_s[:, :] = vcnt_s[:, :] + jnp.dot(ohT, ones, preferred_element_type=_f32)

    blo = bsm[0, 0, 0]
    bhi = bsm[0, 0, VBLK - 1]
    for j in range(B):
        @pl.when((j >= blo) & (j <= bhi))
        def _upd(j=j):
            mv = bcol == float(j)
            emin_s[j:j + 1, :] = jnp.minimum(
                emin_s[j:j + 1, :], jnp.min(jnp.where(mv, mn, jnp.inf), axis=0, keepdims=True))
            emax_s[j:j + 1, :] = jnp.maximum(
                emax_s[j:j + 1, :], jnp.max(jnp.where(mv, mx, -jnp.inf), axis=0, keepdims=True))
            vmin_s[j:j + 1, :] = jnp.minimum(
                vmin_s[j:j + 1, :], jnp.min(jnp.where(mv, vo, jnp.inf), axis=0, keepdims=True))
            vmax_s[j:j + 1, :] = jnp.maximum(
                vmax_s[j:j + 1, :], jnp.max(jnp.where(mv, vo, -jnp.inf), axis=0, keepdims=True))

    @pl.when(pid == nb - 1)
    def _final():
        ec = ecnt_s[:, 0:1]
        vc = vcnt_s[:, 0:1]
        emn = jnp.where(ec > 0.0, emin_s[:, :], 0.0)
        emx = jnp.where(ec > 0.0, emax_s[:, :], 0.0)
        eme = esum_s[:, :] * (1.0 / jnp.maximum(ec, 1.0))
        vmn = jnp.where(vc > 0.0, vmin_s[:, :], 0.0)
        vmx = jnp.where(vc > 0.0, vmax_s[:, :], 0.0)
        vme = vsum_s[:, :] * (1.0 / jnp.maximum(vc, 1.0))
        dot = lambda a, w: jnp.dot(a, w[:, :], preferred_element_type=_f32)
        z = (dot(gg[:, :], wgg) + dot(emn, wem) + dot(eme, wee)
             + dot(esum_s[:, :], wes) + dot(emx, wex) + dot(vmn, wvm)
             + dot(vme, wve) + dot(vsum_s[:, :], wvs) + dot(vmx, wvx)
             + bg1r[:, :])
        out_o[:, :] = dot(jnp.maximum(z, 0.0), wg2) + bg2r[:, :]


def _k5(MN, S, MX, CNT, v_out, batchcol, batch3, g, Wg_slices, bg1, Wg2, bg2):
    nb = N // VBLK
    return pl.pallas_call(
        _k5_body,
        grid=(nb,),
        in_specs=[
            pl.BlockSpec((VBLK, OE), lambda i: (i, 0)),
            pl.BlockSpec((VBLK, OE), lambda i: (i, 0)),
            pl.BlockSpec((VBLK, OE), lambda i: (i, 0)),
            pl.BlockSpec((VBLK, 16), lambda i: (i, 0)),
            pl.BlockSpec((VBLK, OV), lambda i: (i, 0)),
            pl.BlockSpec((VBLK, 1), lambda i: (i, 0)),
            pl.BlockSpec((1, 1, VBLK), lambda i: (i, 0, 0)),
            pl.BlockSpec((1, 1, VBLK), lambda i: (i, 0, 0), memory_space=pltpu.SMEM),
            _full((B, DG)),
            _full((DG, HG)), _full((OE, HG)), _full((OE, HG)), _full((OE, HG)),
            _full((OE, HG)), _full((OV, HG)), _full((OV, HG)), _full((OV, HG)),
            _full((OV, HG)), _full((HG, OG)), _full((1, HG)), _full((1, OG)),
        ],
        out_specs=pl.BlockSpec((B, OG), lambda i: (0, 0)),
        out_shape=jax.ShapeDtypeStruct((B, OG), _f32),
        scratch_shapes=[
            pltpu.VMEM((B, OE), _f32), pltpu.VMEM((B, OE), _f32),
            pltpu.VMEM((B, OE), _f32), pltpu.VMEM((B, 16), _f32),
            pltpu.VMEM((B, OV), _f32), pltpu.VMEM((B, OV), _f32),
            pltpu.VMEM((B, OV), _f32), pltpu.VMEM((B, 16), _f32),
        ],
    )(MN, S, MX, CNT, v_out, batchcol, batch3, batch3, g, *Wg_slices, Wg2, bg1, bg2)


def kernel(v_attr, e_attr, g, We1, be1, We2, be2, Wv1, bv1, Wv2, bv2,
           Wg1, bg1, Wg2, bg2, edgeij_pair, batch):
    src = edgeij_pair[0].astype(_i32)
    dst = edgeij_pair[1].astype(_i32)
    batch = batch.astype(_i32)
    batchcol = batch.astype(_f32).reshape(N, 1)
    batch3 = batch.reshape(N // VBLK, 1, VBLK)

    A, Bm, Ce, Cg = We1[0:256], We1[256:512], We1[512:528], We1[528:592]
    Av, Wmn, Wme, Wsm, Wmx, Cv = (Wv1[0:256], Wv1[256:512], Wv1[512:768],
                                  Wv1[768:1024], Wv1[1024:1280], Wv1[1280:1344])
    Wg_slices = (Wg1[0:64], Wg1[64:320], Wg1[320:576], Wg1[576:832],
                 Wg1[832:1088], Wg1[1088:1344], Wg1[1344:1600],
                 Wg1[1600:1856], Wg1[1856:2112])
    be1r, bv1r, be2r, bv2r = (be1.reshape(1, HE), bv1.reshape(1, HV),
                              be2.reshape(1, OE), bv2.reshape(1, OV))
    bg1r, bg2r = bg1.reshape(1, HG), bg2.reshape(1, OG)

    P2, Q, V1 = _k2(v_attr, batchcol, g, A, Bm, Av, Cg, Cv, be1r, bv1r)

    Zg = _sc_gather_add(P2, Q, src, dst)

    e_out = _k3(Zg, e_attr, Ce, We2, be2r)

    S, MN, MX, CNT = _k6(e_out, src.reshape(E // EBLK6, 1, EBLK6))

    v_out = _k4(V1, MN, S, MX, CNT, Wmn, Wme, Wsm, Wmx, Wv2, bv2r)

    g_out = _k5(MN, S, MX, CNT, v_out, batchcol, batch3, g, Wg_slices,
                bg1r, Wg2, bg2r)

    return (e_out, v_out, g_out)
